# baseline (speedup 1.0000x reference)
import functools

import jax
import jax.numpy as jnp
from jax import lax
from jax.experimental import pallas as pl
from jax.experimental.pallas import tpu as pltpu
from jax.experimental.pallas import tpu_sc as plsc

_B, _N, _F = 4096, 32, 256
_NC, _NS, _L = 2, 16, 16
_NW = _NC * _NS
_RPW = _B // _NW
_COL0 = 2 * _F

_mesh = plsc.VectorSubcoreMesh(core_axis_name="c", subcore_axis_name="s")


@functools.partial(
    pl.kernel,
    mesh=_mesh,
    out_type=jax.ShapeDtypeStruct((_B, 2), jnp.float32),
    scratch_types=[
        pltpu.VMEM((_RPW, _L), jnp.float32),
        pltpu.VMEM((_RPW, 2), jnp.float32),
    ],
    compiler_params=pltpu.CompilerParams(
        use_tc_tiling_on_sc=False, needs_layout_passes=False
    ),
)
def _person_rule_sc(x_hbm, out_hbm, buf_v, y_v):
    wid = lax.axis_index("s") * _NC + lax.axis_index("c")
    base = wid * _RPW
    pltpu.sync_copy(x_hbm.at[pl.ds(base, _RPW), 2, pl.ds(0, _L)], buf_v)
    iota = lax.broadcasted_iota(jnp.int32, (_L,), 0)
    zeros = jnp.zeros((_L,), jnp.int32)
    ones = jnp.ones((_L,), jnp.int32)
    for i in range(_RPW // _L):
        ridx = iota + (i * _L)
        v0 = plsc.load_gather(buf_v, [ridx, zeros])
        v1 = plsc.load_gather(buf_v, [ridx, ones])
        t0 = jnp.where(v0 > 0, 1.0, v0)
        t1 = jnp.where(v1 > 0, 1.0, v1)
        zb = t0 + t1
        y0 = jnp.where(zb == 0, 100.0, -100.0)
        y1 = jnp.where(zb > 0, 100.0, -100.0)
        plsc.store_scatter(y_v, [ridx, zeros], y0)
        plsc.store_scatter(y_v, [ridx, ones], y1)
    pltpu.sync_copy(y_v, out_hbm.at[pl.ds(base, _RPW)])


def kernel(x, adj_mat):
    del adj_mat
    return _person_rule_sc(x)


# speedup vs baseline: 4.6785x; 4.6785x over previous
import functools

import jax
import jax.numpy as jnp
from jax import lax
from jax.experimental import pallas as pl
from jax.experimental.pallas import tpu as pltpu
from jax.experimental.pallas import tpu_sc as plsc

_B, _N, _F = 4096, 32, 256
_NC, _NS, _L = 2, 16, 16
_NW = _NC * _NS
_RPW = _B // _NW

_mesh = plsc.VectorSubcoreMesh(core_axis_name="c", subcore_axis_name="s")


@functools.partial(
    pl.kernel,
    mesh=_mesh,
    out_type=jax.ShapeDtypeStruct((_B, 2), jnp.float32),
    scratch_types=[
        pltpu.VMEM((_RPW,), jnp.int32),
        pltpu.VMEM((_RPW, _F), jnp.float32),
        pltpu.VMEM((_RPW, 2), jnp.float32),
        pltpu.SemaphoreType.DMA,
    ],
    compiler_params=pltpu.CompilerParams(needs_layout_passes=False),
)
def _person_rule_sc(x_hbm, out_hbm, idx_v, rows_v, y_v, sem):
    wid = lax.axis_index("s") * _NC + lax.axis_index("c")
    base = wid * _RPW
    iota = lax.broadcasted_iota(jnp.int32, (_L,), 0)
    for i in range(_RPW // _L):
        idx_v[pl.ds(i * _L, _L)] = (base + i * _L + iota) * _N + 2
    pltpu.async_copy(x_hbm.at[idx_v], rows_v, sem).wait()
    zeros = jnp.zeros((_L,), jnp.int32)
    ones = jnp.ones((_L,), jnp.int32)
    for i in range(_RPW // _L):
        ridx = iota + (i * _L)
        v0 = plsc.load_gather(rows_v, [ridx, zeros])
        v1 = plsc.load_gather(rows_v, [ridx, ones])
        t0 = jnp.where(v0 > 0, 1.0, v0)
        t1 = jnp.where(v1 > 0, 1.0, v1)
        zb = t0 + t1
        y0 = jnp.where(zb == 0, 100.0, -100.0)
        y1 = jnp.where(zb > 0, 100.0, -100.0)
        plsc.store_scatter(y_v, [ridx, zeros], y0)
        plsc.store_scatter(y_v, [ridx, ones], y1)
    pltpu.sync_copy(y_v, out_hbm.at[pl.ds(base, _RPW)])


def kernel(x, adj_mat):
    del adj_mat
    return _person_rule_sc(x.reshape(_B * _N, _F))
